# baseline (speedup 1.0000x reference)
import functools

import jax
import jax.numpy as jnp
from jax import lax
from jax.experimental import pallas as pl
from jax.experimental.pallas import tpu as pltpu
from jax.experimental.pallas import tpu_sc as plsc

BATCH = 4096
HIST = 200
EMBED_DIM = 64
NUM_CLASSES = 100

NUM_CORES = 2
NUM_SUBCORES = 16
NUM_WORKERS = NUM_CORES * NUM_SUBCORES
ROWS_PER_WORKER = BATCH // NUM_WORKERS
SPLIT0 = 104
SPLIT1 = HIST - SPLIT0


def _pool_kernel(x_hbm, table_hbm, out_hbm, idx_v, rows_v, pooled_v,
                 sem0, sem1):
    wid = lax.axis_index("s") * NUM_CORES + lax.axis_index("c")
    base = wid * ROWS_PER_WORKER

    pltpu.sync_copy(x_hbm.at[pl.ds(base, ROWS_PER_WORKER)], idx_v)

    sems = (sem0, sem1)

    def row_copies(r, buf):
        return (
            pltpu.make_async_copy(
                table_hbm.at[idx_v.at[r, pl.ds(0, SPLIT0)]],
                rows_v.at[buf, pl.ds(0, SPLIT0)],
                sems[buf],
            ),
            pltpu.make_async_copy(
                table_hbm.at[idx_v.at[r, pl.ds(SPLIT0, SPLIT1)]],
                rows_v.at[buf, pl.ds(SPLIT0, SPLIT1)],
                sems[buf],
            ),
        )

    def fire(r, buf):
        for cp in row_copies(r, buf):
            cp.start()

    def wait(r, buf):
        for cp in row_copies(r, buf):
            cp.wait()

    def accumulate(r, buf):
        zeros = jnp.zeros((16,), jnp.float32)

        def body(j, accs):
            return tuple(
                accs[c] + rows_v[buf, j, pl.ds(c * 16, 16)]
                for c in range(EMBED_DIM // 16)
            )

        accs = lax.fori_loop(0, HIST, body, (zeros,) * (EMBED_DIM // 16))
        for c in range(EMBED_DIM // 16):
            pooled_v[r, pl.ds(c * 16, 16)] = accs[c]

    fire(0, 0)

    def outer(r2, carry):
        r = r2 * 2
        fire(r + 1, 1)
        wait(r, 0)
        accumulate(r, 0)

        @pl.when(r + 2 < ROWS_PER_WORKER)
        def _():
            fire(r + 2, 0)

        wait(r + 1, 1)
        accumulate(r + 1, 1)
        return carry

    lax.fori_loop(0, ROWS_PER_WORKER // 2, outer, 0)

    pltpu.sync_copy(pooled_v, out_hbm.at[pl.ds(base, ROWS_PER_WORKER)])


def _pooled_sums(x, table):
    mesh = plsc.VectorSubcoreMesh(
        core_axis_name="c", subcore_axis_name="s",
        num_cores=NUM_CORES, num_subcores=NUM_SUBCORES)
    f = pl.kernel(
        _pool_kernel,
        out_type=jax.ShapeDtypeStruct((BATCH, EMBED_DIM), jnp.float32),
        mesh=mesh,
        scratch_types=[
            pltpu.VMEM((ROWS_PER_WORKER, HIST), jnp.int32),
            pltpu.VMEM((2, HIST, EMBED_DIM), jnp.float32),
            pltpu.VMEM((ROWS_PER_WORKER, EMBED_DIM), jnp.float32),
            pltpu.SemaphoreType.DMA,
            pltpu.SemaphoreType.DMA,
        ],
        compiler_params=pltpu.CompilerParams(use_tc_tiling_on_sc=False),
    )
    return f(x, table)


def _head_kernel(p_ref, wt_ref, b_ref, out_ref):
    out_ref[...] = (
        jnp.dot(p_ref[...], wt_ref[...],
                preferred_element_type=jnp.float32)
        + b_ref[...]
    )


def _classifier_head(pooled, Wt_pad, b_pad, n_pad):
    blk = 512
    return pl.pallas_call(
        _head_kernel,
        grid=(BATCH // blk,),
        in_specs=[
            pl.BlockSpec((blk, EMBED_DIM), lambda i: (i, 0)),
            pl.BlockSpec((EMBED_DIM, n_pad), lambda i: (0, 0)),
            pl.BlockSpec((1, n_pad), lambda i: (0, 0)),
        ],
        out_specs=pl.BlockSpec((blk, n_pad), lambda i: (i, 0)),
        out_shape=jax.ShapeDtypeStruct((BATCH, n_pad), jnp.float32),
    )(pooled, Wt_pad, b_pad)


@jax.jit
def kernel(x, table, W, b):
    x = x.astype(jnp.int32)
    pooled = _pooled_sums(x, table)

    n_pad = 128
    Wt_pad = jnp.zeros((EMBED_DIM, n_pad), jnp.float32)
    Wt_pad = Wt_pad.at[:, :NUM_CLASSES].set(W.T)
    b_pad = jnp.zeros((1, n_pad), jnp.float32).at[0, :NUM_CLASSES].set(b)

    out = _classifier_head(pooled, Wt_pad, b_pad, n_pad)
    return out[:, :NUM_CLASSES]


# speedup vs baseline: 1.0157x; 1.0157x over previous
import functools

import jax
import jax.numpy as jnp
from jax import lax
from jax.experimental import pallas as pl
from jax.experimental.pallas import tpu as pltpu
from jax.experimental.pallas import tpu_sc as plsc

BATCH = 4096
HIST = 200
EMBED_DIM = 64
NUM_CLASSES = 100

NUM_CORES = 2
NUM_SUBCORES = 16
NUM_WORKERS = NUM_CORES * NUM_SUBCORES
ROWS_PER_WORKER = BATCH // NUM_WORKERS
SPLIT0 = 104
SPLIT1 = HIST - SPLIT0


def _pool_kernel(x_hbm, table_hbm, out_hbm, idx_v, rows_v, pooled_v,
                 sem0, sem1):
    wid = lax.axis_index("s") * NUM_CORES + lax.axis_index("c")
    base = wid * ROWS_PER_WORKER

    pltpu.sync_copy(x_hbm.at[pl.ds(base * HIST, ROWS_PER_WORKER * HIST)],
                    idx_v)

    sems = (sem0, sem1)

    def row_copies(r, buf):
        return (
            pltpu.make_async_copy(
                table_hbm.at[idx_v.at[pl.ds(r * HIST, SPLIT0)]],
                rows_v.at[buf, pl.ds(0, SPLIT0)],
                sems[buf],
            ),
            pltpu.make_async_copy(
                table_hbm.at[idx_v.at[pl.ds(r * HIST + SPLIT0, SPLIT1)]],
                rows_v.at[buf, pl.ds(SPLIT0, SPLIT1)],
                sems[buf],
            ),
        )

    def fire(r, buf):
        for cp in row_copies(r, buf):
            cp.start()

    def wait(r, buf):
        for cp in row_copies(r, buf):
            cp.wait()

    def accumulate(r, buf):
        zeros = jnp.zeros((16,), jnp.float32)
        unroll = 4

        def body(j4, accs):
            j = j4 * unroll
            accs = list(accs)
            for u in range(unroll):
                for c in range(EMBED_DIM // 16):
                    accs[c] = accs[c] + rows_v[buf, j + u, pl.ds(c * 16, 16)]
            return tuple(accs)

        accs = lax.fori_loop(0, HIST // unroll, body,
                             (zeros,) * (EMBED_DIM // 16))
        for c in range(EMBED_DIM // 16):
            pooled_v[r, pl.ds(c * 16, 16)] = accs[c]

    fire(0, 0)

    def outer(r2, carry):
        r = r2 * 2
        fire(r + 1, 1)
        wait(r, 0)
        accumulate(r, 0)

        @pl.when(r + 2 < ROWS_PER_WORKER)
        def _():
            fire(r + 2, 0)

        wait(r + 1, 1)
        accumulate(r + 1, 1)
        return carry

    lax.fori_loop(0, ROWS_PER_WORKER // 2, outer, 0)

    pltpu.sync_copy(pooled_v, out_hbm.at[pl.ds(base, ROWS_PER_WORKER)])


def _pooled_sums(x, table):
    mesh = plsc.VectorSubcoreMesh(
        core_axis_name="c", subcore_axis_name="s",
        num_cores=NUM_CORES, num_subcores=NUM_SUBCORES)
    f = pl.kernel(
        _pool_kernel,
        out_type=jax.ShapeDtypeStruct((BATCH, EMBED_DIM), jnp.float32),
        mesh=mesh,
        scratch_types=[
            pltpu.VMEM((ROWS_PER_WORKER * HIST,), jnp.int32),
            pltpu.VMEM((2, HIST, EMBED_DIM), jnp.float32),
            pltpu.VMEM((ROWS_PER_WORKER, EMBED_DIM), jnp.float32),
            pltpu.SemaphoreType.DMA,
            pltpu.SemaphoreType.DMA,
        ],
        compiler_params=pltpu.CompilerParams(use_tc_tiling_on_sc=False),
    )
    return f(x.reshape(-1), table)


def _head_kernel(p_ref, wt_ref, b_ref, out_ref):
    out_ref[...] = (
        jnp.dot(p_ref[...], wt_ref[...],
                preferred_element_type=jnp.float32)
        + b_ref[...]
    )


def _classifier_head(pooled, Wt_pad, b_pad, n_pad):
    blk = 512
    return pl.pallas_call(
        _head_kernel,
        grid=(BATCH // blk,),
        in_specs=[
            pl.BlockSpec((blk, EMBED_DIM), lambda i: (i, 0)),
            pl.BlockSpec((EMBED_DIM, n_pad), lambda i: (0, 0)),
            pl.BlockSpec((1, n_pad), lambda i: (0, 0)),
        ],
        out_specs=pl.BlockSpec((blk, n_pad), lambda i: (i, 0)),
        out_shape=jax.ShapeDtypeStruct((BATCH, n_pad), jnp.float32),
    )(pooled, Wt_pad, b_pad)


@jax.jit
def kernel(x, table, W, b):
    x = x.astype(jnp.int32)
    pooled = _pooled_sums(x, table)

    n_pad = 128
    Wt_pad = jnp.zeros((EMBED_DIM, n_pad), jnp.float32)
    Wt_pad = Wt_pad.at[:, :NUM_CLASSES].set(W.T)
    b_pad = jnp.zeros((1, n_pad), jnp.float32).at[0, :NUM_CLASSES].set(b)

    out = _classifier_head(pooled, Wt_pad, b_pad, n_pad)
    return out[:, :NUM_CLASSES]
